# baseline (speedup 1.0000x reference)
import functools

import jax
import jax.numpy as jnp
from jax import lax
from jax.experimental import pallas as pl
from jax.experimental.pallas import tpu as pltpu
from jax.experimental.pallas import tpu_sc as plsc

B, T, H = 1024, 50, 128
BT = B * T
KV = 1000
OV = 1000
SV = 100000
NC, NS = 2, 16
NW = NC * NS

KCH = 64
KNCH = (4 * T * B) // KCH // NW
KSLOT = 5
OCH = 64
ROWS_W = BT // NW
ONCH = ROWS_W // OCH
SROWS = B // NW


MESH = plsc.VectorSubcoreMesh(core_axis_name="c", subcore_axis_name="s")


def _sc_gather_obs(ocat_f, scat_f, o_tab, s_tab):
    @functools.partial(
        pl.kernel,
        out_type=(
            jax.ShapeDtypeStruct((BT, 2 * H), jnp.float32),
            jax.ShapeDtypeStruct((6 * B, H), jnp.float32),
        ),
        mesh=MESH,
        scratch_types=[
            pltpu.VMEM((2, OCH), jnp.int32),
            pltpu.VMEM((2, OCH, H), jnp.float32),
            pltpu.VMEM((SROWS,), jnp.int32),
            pltpu.VMEM((SROWS, H), jnp.float32),
            pltpu.SemaphoreType.DMA,
        ],
    )
    def body(ocat_hbm, scat_hbm, otab_hbm, stab_hbm,
             oout_hbm, sout_hbm,
             oidx_v, orows_v, sidx_v, srows_v, sem):
        wid = lax.axis_index("s") * NC + lax.axis_index("c")

        sbase = wid * SROWS
        for i in range(2):
            pltpu.sync_copy(scat_hbm.at[pl.ds(i * B + sbase, SROWS)], sidx_v)
            if i:
                for v in range(SROWS // 16):
                    sl = pl.ds(v * 16, 16)
                    sidx_v[sl] = sidx_v[sl] + i * SV
            pltpu.async_copy(stab_hbm.at[sidx_v], srows_v, sem).wait()
            pltpu.sync_copy(srows_v, sout_hbm.at[pl.ds(i * B + sbase, SROWS)])

        def oblock(c, carry):
            base = wid * ROWS_W + c * OCH
            for g in range(2):
                pltpu.sync_copy(
                    ocat_hbm.at[pl.ds(g * BT + base, OCH)], oidx_v.at[g])
            for q in range(OCH // 16):
                sl = pl.ds(q * 16, 16)
                oidx_v[1, sl] = oidx_v[1, sl] + OV
            descs = [
                pltpu.async_copy(otab_hbm.at[oidx_v.at[g]], orows_v.at[g], sem)
                for g in range(2)
            ]
            for d in descs:
                d.wait()
            descs = [
                pltpu.async_copy(
                    orows_v.at[g],
                    oout_hbm.at[pl.ds(base, OCH), pl.ds(g * H, H)], sem)
                for g in range(2)
            ]
            for d in descs:
                d.wait()
            return carry

        lax.fori_loop(0, ONCH, oblock, 0)

    return body(ocat_f, scat_f, o_tab, s_tab)


def _sc_gather_known(kcat_f, k_tab):
    @functools.partial(
        pl.kernel,
        out_type=jax.ShapeDtypeStruct((T * 12 * B, H), jnp.float32),
        mesh=MESH,
        scratch_types=[
            pltpu.VMEM((KSLOT, KCH), jnp.int32),
            pltpu.VMEM((KSLOT, KCH, H), jnp.float32),
            pltpu.SemaphoreType.DMA,
        ],
    )
    def body(kcat_hbm, ktab_hbm, kout_hbm, kidx_v, krows_v, sem):
        wid = lax.axis_index("s") * NC + lax.axis_index("c")
        kv = wid // (NW // 4)
        koff = kv * KV
        rbase = (wid % (NW // 4)) * KNCH

        def kblock(jb, carry):
            c0 = jb * KSLOT
            for u in range(KSLOT):
                src = (wid * KNCH + c0 + u) * KCH
                pltpu.sync_copy(kcat_hbm.at[pl.ds(src, KCH)], kidx_v.at[u])
                for q in range(KCH // 16):
                    sl = pl.ds(q * 16, 16)
                    kidx_v[u, sl] = kidx_v[u, sl] + koff
            descs = [
                pltpu.async_copy(ktab_hbm.at[kidx_v.at[u]], krows_v.at[u], sem)
                for u in range(KSLOT)
            ]
            for d in descs:
                d.wait()
            for u in range(KSLOT):
                r = rbase + c0 + u
                t = r // (B // KCH)
                bc = r % (B // KCH)
                dst = (t * 12 + kv) * B + bc * KCH
                pltpu.sync_copy(krows_v.at[u], kout_hbm.at[pl.ds(dst, KCH)])
            return carry

        lax.fori_loop(0, KNCH // KSLOT, kblock, 0)

    return body(kcat_f, k_tab)


def _known_cont_body(c_ref, vec_ref, bias_ref, alias_ref, out_ref):
    j = pl.program_id(1)
    c = c_ref[0]
    vec = vec_ref[...]
    bias = bias_ref[...]
    for jj in range(2):

        @pl.when(j == jj)
        def _():
            for u in range(4):
                cv = 4 * jj + u
                out_ref[u * B:(u + 1) * B, :] = jnp.dot(
                    c[:, cv:cv + 1], vec[cv:cv + 1, :],
                    preferred_element_type=jnp.float32) + bias[cv:cv + 1, :]


def _known_cont_fill(cont_tb, vec, bias, cat_buf):
    return pl.pallas_call(
        _known_cont_body,
        grid=(T, 2),
        in_specs=[
            pl.BlockSpec((1, B, 8), lambda t, j: (t, 0, 0)),
            pl.BlockSpec((8, H), lambda t, j: (0, 0)),
            pl.BlockSpec((8, H), lambda t, j: (0, 0)),
            pl.BlockSpec((8, H), lambda t, j: (0, 0)),
        ],
        out_specs=pl.BlockSpec((4 * B, H), lambda t, j: (3 * t + 1 + j, 0)),
        out_shape=jax.ShapeDtypeStruct((T * 12 * B, H), jnp.float32),
        input_output_aliases={3: 0},
    )(cont_tb, vec, bias, cat_buf)


GBO = 32


def _obs_asm_body(cat_ref, c_ref, vec_ref, bias_ref, out_ref):
    c = c_ref[...]
    vec = vec_ref[...]
    bias = bias_ref[...]
    for v in range(2):
        out_ref[:, :, v, :] = cat_ref[:, v * H:(v + 1) * H].reshape(GBO, T, H)
    for cv in range(6):
        out_ref[:, :, 2 + cv, :] = (jnp.dot(
            c[:, cv:cv + 1], vec[cv:cv + 1, :],
            preferred_element_type=jnp.float32) + bias[cv:cv + 1, :]
        ).reshape(GBO, T, H)


def _obs_assemble(cat_buf, cont2, vec, bias):
    return pl.pallas_call(
        _obs_asm_body,
        grid=(B // GBO,),
        in_specs=[
            pl.BlockSpec((GBO * T, 2 * H), lambda i: (i, 0)),
            pl.BlockSpec((GBO * T, 6), lambda i: (i, 0)),
            pl.BlockSpec((6, H), lambda i: (0, 0)),
            pl.BlockSpec((6, H), lambda i: (0, 0)),
        ],
        out_specs=pl.BlockSpec((GBO, T, 8, H), lambda i: (i, 0, 0, 0)),
        out_shape=jax.ShapeDtypeStruct((B, T, 8, H), jnp.float32),
    )(cat_buf, cont2, vec, bias)


def _static_cont_body(c_ref, vec_ref, bias_ref, alias_ref, out_ref):
    v = pl.program_id(0)
    c = c_ref[...]
    vec = vec_ref[...]
    bias = bias_ref[...]
    for vv in range(4):

        @pl.when(v == vv)
        def _():
            out_ref[...] = jnp.dot(
                c[:, vv:vv + 1], vec[vv:vv + 1, :],
                preferred_element_type=jnp.float32) + bias[vv:vv + 1, :]


def _static_cont_fill(cont2, vec, bias, cat_buf):
    return pl.pallas_call(
        _static_cont_body,
        grid=(4,),
        in_specs=[
            pl.BlockSpec((B, 4), lambda v: (0, 0)),
            pl.BlockSpec((4, H), lambda v: (0, 0)),
            pl.BlockSpec((4, H), lambda v: (0, 0)),
            pl.BlockSpec((8, H), lambda v: (0, 0)),
        ],
        out_specs=pl.BlockSpec((B, H), lambda v: (v + 2, 0)),
        out_shape=jax.ShapeDtypeStruct((6 * B, H), jnp.float32),
        input_output_aliases={3: 0},
    )(cont2, vec, bias, cat_buf)


GBT = 64


def _tgt_body(c_ref, vec_ref, bias_ref, out_ref):
    out_ref[:, :, 0, :] = (
        c_ref[...] * vec_ref[...][0][None, :] + bias_ref[...][0][None, :]
    ).reshape(GBT, T, H)


def _tgt_fill(cont2, vec, bias):
    return pl.pallas_call(
        _tgt_body,
        grid=(B // GBT,),
        in_specs=[
            pl.BlockSpec((GBT * T, 1), lambda i: (i, 0)),
            pl.BlockSpec((1, H), lambda i: (0, 0)),
            pl.BlockSpec((1, H), lambda i: (0, 0)),
        ],
        out_specs=pl.BlockSpec((GBT, T, 1, H), lambda i: (i, 0, 0, 0)),
        out_shape=jax.ShapeDtypeStruct((B, T, 1, H), jnp.float32),
    )(cont2, vec, bias)


def kernel(s_cat, s_cont, k_cat, k_cont, o_cat, o_cont, target,
           s_cat_tables, k_cat_tables, o_cat_tables,
           s_cont_vec, s_cont_bias, k_cont_vec, k_cont_bias,
           o_cont_vec, o_cont_bias, tgt_vec, tgt_bias):
    kcat_f = jnp.transpose(k_cat, (2, 1, 0)).reshape(-1)
    ocat_f = o_cat.reshape(BT, 2).T.reshape(-1)
    scat_f = s_cat[:, 0, :].T.reshape(-1)
    k_tab = k_cat_tables.reshape(4 * KV, H)
    o_tab = o_cat_tables.reshape(2 * OV, H)
    s_tab = s_cat_tables.reshape(2 * SV, H)

    obuf, sbuf = _sc_gather_obs(ocat_f, scat_f, o_tab, s_tab)

    kc_tb = jnp.transpose(k_cont, (1, 0, 2))
    t_full = _tgt_fill(target.reshape(BT, 1), tgt_vec, tgt_bias)

    kbuf = _sc_gather_known(kcat_f, k_tab)

    o_full = _obs_assemble(obuf, o_cont.reshape(BT, 6),
                           o_cont_vec, o_cont_bias)
    sbuf = _static_cont_fill(s_cont[:, 0, :],
                             s_cont_vec, s_cont_bias, sbuf)
    kbuf = _known_cont_fill(kc_tb, k_cont_vec, k_cont_bias, kbuf)

    k_full = jnp.transpose(kbuf.reshape(T, 12, B, H), (2, 0, 1, 3))
    s_full = jnp.transpose(sbuf.reshape(6, B, H), (1, 0, 2))
    return (s_full, k_full, o_full, t_full)


# speedup vs baseline: 1.0126x; 1.0126x over previous
import functools

import jax
import jax.numpy as jnp
from jax import lax
from jax.experimental import pallas as pl
from jax.experimental.pallas import tpu as pltpu
from jax.experimental.pallas import tpu_sc as plsc

B, T, H = 1024, 50, 128
BT = B * T
KV = 1000
OV = 1000
SV = 100000
NC, NS = 2, 16
NW = NC * NS

KCH = 64
KNCH = (4 * T * B) // KCH // NW
KSLOT = 5
OCH = 64
ROWS_W = BT // NW
ONCH = ROWS_W // OCH
SROWS = B // NW


MESH = plsc.VectorSubcoreMesh(core_axis_name="c", subcore_axis_name="s")


def _sc_gather_obs(ocat_f, scat_f, o_tab, s_tab):
    @functools.partial(
        pl.kernel,
        out_type=(
            jax.ShapeDtypeStruct((BT, 2 * H), jnp.float32),
            jax.ShapeDtypeStruct((6 * B, H), jnp.float32),
        ),
        mesh=MESH,
        scratch_types=[
            pltpu.VMEM((2, OCH), jnp.int32),
            pltpu.VMEM((2, OCH, H), jnp.float32),
            pltpu.VMEM((SROWS,), jnp.int32),
            pltpu.VMEM((SROWS, H), jnp.float32),
            pltpu.SemaphoreType.DMA,
        ],
    )
    def body(ocat_hbm, scat_hbm, otab_hbm, stab_hbm,
             oout_hbm, sout_hbm,
             oidx_v, orows_v, sidx_v, srows_v, sem):
        wid = lax.axis_index("s") * NC + lax.axis_index("c")

        sbase = wid * SROWS
        for i in range(2):
            pltpu.sync_copy(scat_hbm.at[pl.ds(i * B + sbase, SROWS)], sidx_v)
            if i:
                for v in range(SROWS // 16):
                    sl = pl.ds(v * 16, 16)
                    sidx_v[sl] = sidx_v[sl] + i * SV
            pltpu.async_copy(stab_hbm.at[sidx_v], srows_v, sem).wait()
            pltpu.sync_copy(srows_v, sout_hbm.at[pl.ds(i * B + sbase, SROWS)])

        def oblock(c, carry):
            base = wid * ROWS_W + c * OCH
            for g in range(2):
                pltpu.sync_copy(
                    ocat_hbm.at[pl.ds(g * BT + base, OCH)], oidx_v.at[g])
            for q in range(OCH // 16):
                sl = pl.ds(q * 16, 16)
                oidx_v[1, sl] = oidx_v[1, sl] + OV
            descs = [
                pltpu.async_copy(otab_hbm.at[oidx_v.at[g]], orows_v.at[g], sem)
                for g in range(2)
            ]
            for d in descs:
                d.wait()
            descs = [
                pltpu.async_copy(
                    orows_v.at[g],
                    oout_hbm.at[pl.ds(base, OCH), pl.ds(g * H, H)], sem)
                for g in range(2)
            ]
            for d in descs:
                d.wait()
            return carry

        lax.fori_loop(0, ONCH, oblock, 0)

    return body(ocat_f, scat_f, o_tab, s_tab)


def _sc_gather_known(kcat_f, k_tab):
    @functools.partial(
        pl.kernel,
        out_type=jax.ShapeDtypeStruct((T * 12 * B, H), jnp.float32),
        mesh=MESH,
        scratch_types=[
            pltpu.VMEM((KSLOT, KCH), jnp.int32),
            pltpu.VMEM((KSLOT, KCH, H), jnp.float32),
            pltpu.SemaphoreType.DMA,
        ],
    )
    def body(kcat_hbm, ktab_hbm, kout_hbm, kidx_v, krows_v, sem):
        wid = lax.axis_index("s") * NC + lax.axis_index("c")
        kv = wid // (NW // 4)
        koff = kv * KV
        rbase = (wid % (NW // 4)) * KNCH

        def kblock(jb, carry):
            c0 = jb * KSLOT
            for u in range(KSLOT):
                src = (wid * KNCH + c0 + u) * KCH
                pltpu.sync_copy(kcat_hbm.at[pl.ds(src, KCH)], kidx_v.at[u])
                for q in range(KCH // 16):
                    sl = pl.ds(q * 16, 16)
                    kidx_v[u, sl] = kidx_v[u, sl] + koff
            descs = [
                pltpu.async_copy(ktab_hbm.at[kidx_v.at[u]], krows_v.at[u], sem)
                for u in range(KSLOT)
            ]
            for d in descs:
                d.wait()
            for u in range(KSLOT):
                r = rbase + c0 + u
                t = r // (B // KCH)
                bc = r % (B // KCH)
                dst = (t * 12 + kv) * B + bc * KCH
                pltpu.sync_copy(krows_v.at[u], kout_hbm.at[pl.ds(dst, KCH)])
            return carry

        lax.fori_loop(0, KNCH // KSLOT, kblock, 0)

    return body(kcat_f, k_tab)


def _known_cont_body(c_ref, vec_ref, bias_ref, alias_ref, out_ref):
    j = pl.program_id(1)
    c = c_ref[0]
    vec = vec_ref[...]
    bias = bias_ref[...]
    for jj in range(2):

        @pl.when(j == jj)
        def _():
            for u in range(4):
                cv = 4 * jj + u
                out_ref[u * B:(u + 1) * B, :] = jnp.dot(
                    c[:, cv:cv + 1], vec[cv:cv + 1, :],
                    preferred_element_type=jnp.float32) + bias[cv:cv + 1, :]


def _known_cont_fill(cont_tb, vec, bias, cat_buf):
    return pl.pallas_call(
        _known_cont_body,
        grid=(T, 2),
        in_specs=[
            pl.BlockSpec((1, B, 8), lambda t, j: (t, 0, 0)),
            pl.BlockSpec((8, H), lambda t, j: (0, 0)),
            pl.BlockSpec((8, H), lambda t, j: (0, 0)),
            pl.BlockSpec((8, H), lambda t, j: (0, 0)),
        ],
        out_specs=pl.BlockSpec((4 * B, H), lambda t, j: (3 * t + 1 + j, 0)),
        out_shape=jax.ShapeDtypeStruct((T * 12 * B, H), jnp.float32),
        input_output_aliases={3: 0},
    )(cont_tb, vec, bias, cat_buf)


GBO = 16


def _obs_asm_body(cat_ref, c_ref, vec_ref, bias_ref, out_ref):
    c = c_ref[...]
    vec = vec_ref[...]
    bias = bias_ref[...]
    for v in range(2):
        out_ref[:, :, v, :] = cat_ref[:, v * H:(v + 1) * H].reshape(GBO, T, H)
    for cv in range(6):
        out_ref[:, :, 2 + cv, :] = (
            c[:, cv:cv + 1] * vec[cv][None, :] + bias[cv][None, :]
        ).reshape(GBO, T, H)


def _obs_assemble(cat_buf, cont2, vec, bias):
    return pl.pallas_call(
        _obs_asm_body,
        grid=(B // GBO,),
        in_specs=[
            pl.BlockSpec((GBO * T, 2 * H), lambda i: (i, 0)),
            pl.BlockSpec((GBO * T, 6), lambda i: (i, 0)),
            pl.BlockSpec((6, H), lambda i: (0, 0)),
            pl.BlockSpec((6, H), lambda i: (0, 0)),
        ],
        out_specs=pl.BlockSpec((GBO, T, 8, H), lambda i: (i, 0, 0, 0)),
        out_shape=jax.ShapeDtypeStruct((B, T, 8, H), jnp.float32),
    )(cat_buf, cont2, vec, bias)


def _static_cont_body(c_ref, vec_ref, bias_ref, alias_ref, out_ref):
    v = pl.program_id(0)
    c = c_ref[...]
    vec = vec_ref[...]
    bias = bias_ref[...]
    for vv in range(4):

        @pl.when(v == vv)
        def _():
            out_ref[...] = jnp.dot(
                c[:, vv:vv + 1], vec[vv:vv + 1, :],
                preferred_element_type=jnp.float32) + bias[vv:vv + 1, :]


def _static_cont_fill(cont2, vec, bias, cat_buf):
    return pl.pallas_call(
        _static_cont_body,
        grid=(4,),
        in_specs=[
            pl.BlockSpec((B, 4), lambda v: (0, 0)),
            pl.BlockSpec((4, H), lambda v: (0, 0)),
            pl.BlockSpec((4, H), lambda v: (0, 0)),
            pl.BlockSpec((8, H), lambda v: (0, 0)),
        ],
        out_specs=pl.BlockSpec((B, H), lambda v: (v + 2, 0)),
        out_shape=jax.ShapeDtypeStruct((6 * B, H), jnp.float32),
        input_output_aliases={3: 0},
    )(cont2, vec, bias, cat_buf)


GBT = 32


def _tgt_body(c_ref, vec_ref, bias_ref, out_ref):
    out_ref[:, :, 0, :] = (
        c_ref[...] * vec_ref[...][0][None, :] + bias_ref[...][0][None, :]
    ).reshape(GBT, T, H)


def _tgt_fill(cont2, vec, bias):
    return pl.pallas_call(
        _tgt_body,
        grid=(B // GBT,),
        in_specs=[
            pl.BlockSpec((GBT * T, 1), lambda i: (i, 0)),
            pl.BlockSpec((1, H), lambda i: (0, 0)),
            pl.BlockSpec((1, H), lambda i: (0, 0)),
        ],
        out_specs=pl.BlockSpec((GBT, T, 1, H), lambda i: (i, 0, 0, 0)),
        out_shape=jax.ShapeDtypeStruct((B, T, 1, H), jnp.float32),
    )(cont2, vec, bias)


def kernel(s_cat, s_cont, k_cat, k_cont, o_cat, o_cont, target,
           s_cat_tables, k_cat_tables, o_cat_tables,
           s_cont_vec, s_cont_bias, k_cont_vec, k_cont_bias,
           o_cont_vec, o_cont_bias, tgt_vec, tgt_bias):
    kcat_f = jnp.transpose(k_cat, (2, 1, 0)).reshape(-1)
    ocat_f = o_cat.reshape(BT, 2).T.reshape(-1)
    scat_f = s_cat[:, 0, :].T.reshape(-1)
    k_tab = k_cat_tables.reshape(4 * KV, H)
    o_tab = o_cat_tables.reshape(2 * OV, H)
    s_tab = s_cat_tables.reshape(2 * SV, H)

    obuf, sbuf = _sc_gather_obs(ocat_f, scat_f, o_tab, s_tab)

    kc_tb = jnp.transpose(k_cont, (1, 0, 2))
    t_full = _tgt_fill(target.reshape(BT, 1), tgt_vec, tgt_bias)

    kbuf = _sc_gather_known(kcat_f, k_tab)

    o_full = _obs_assemble(obuf, o_cont.reshape(BT, 6),
                           o_cont_vec, o_cont_bias)
    sbuf = _static_cont_fill(s_cont[:, 0, :],
                             s_cont_vec, s_cont_bias, sbuf)
    kbuf = _known_cont_fill(kc_tb, k_cont_vec, k_cont_bias, kbuf)

    k_full = jnp.transpose(kbuf.reshape(T, 12, B, H), (2, 0, 1, 3))
    s_full = jnp.transpose(sbuf.reshape(6, B, H), (1, 0, 2))
    return (s_full, k_full, o_full, t_full)
